# baseline (speedup 1.0000x reference)
import functools

import jax
import jax.numpy as jnp
from jax import lax
from jax.experimental import pallas as pl
from jax.experimental.pallas import tpu as pltpu
from jax.experimental.pallas import tpu_sc as plsc

B = 4096
F = 26
V = 100000
D = 32
NUM = 13
H1 = 512
H2 = 256
OUT = 100
EPS = 1e-5
NUMP = 128

_NC, _NS = 2, 16
_NW = _NC * _NS
_BT = B * F
_BPW = _BT // _NW
_CH = 256
_NCHUNK = _BPW // _CH
_TBLK = F * V // 4


@functools.cache
def _make_sc_gather():
    mesh = plsc.VectorSubcoreMesh(core_axis_name="c", subcore_axis_name="s")

    @functools.partial(
        pl.kernel,
        mesh=mesh,
        out_type=jax.ShapeDtypeStruct((_BT, 4 * D), jnp.float32),
        scratch_types=[
            pltpu.VMEM((_BPW,), jnp.int32),
            pltpu.VMEM((_CH, 4 * D), jnp.float32),
            pltpu.VMEM((_CH, 4 * D), jnp.float32),
            pltpu.SemaphoreType.DMA,
            pltpu.SemaphoreType.DMA,
            pltpu.SemaphoreType.DMA,
            pltpu.SemaphoreType.DMA,
        ],
    )
    def _sc_gather(table_hbm, idx_hbm, out_hbm, idx_v, r0, r1,
                   sg0, sg1, sw0, sw1):
        wid = lax.axis_index("s") * _NC + lax.axis_index("c")
        base = wid * _BPW
        pltpu.sync_copy(idx_hbm.at[pl.ds(base, _BPW)], idx_v)
        bufs, gsem, wsem = (r0, r1), (sg0, sg1), (sw0, sw1)
        wprev = [None, None]
        for c in range(_NCHUNK):
            b = c & 1
            if wprev[b] is not None:
                wprev[b].wait()
            g = pltpu.async_copy(
                table_hbm.at[idx_v.at[pl.ds(c * _CH, _CH)]], bufs[b], gsem[b])
            g.wait()
            wprev[b] = pltpu.async_copy(
                bufs[b], out_hbm.at[pl.ds(base + c * _CH, _CH)], wsem[b])
        wprev[0].wait()
        wprev[1].wait()

    return _sc_gather


_RSEL = 256


def _select_body(e_ref, s_ref, o_ref):
    e = e_ref[...]
    s = s_ref[...]
    outs = []
    for f in range(F):
        blk = e[:, f * 128:(f + 1) * 128]
        sf = s[:, f:f + 1]
        acc = blk[:, 0:32] * (sf == 0)
        for j in range(1, 4):
            acc = acc + blk[:, j * 32:(j + 1) * 32] * (sf == j)
        outs.append(acc)
    o_ref[...] = jnp.concatenate(outs, axis=1)


def _mlp_body(emb_ref, xn_ref, gn_ref, bn_ref, w1a_ref, w1b_ref, b1_ref,
              g1_ref, be1_ref, w2_ref, b2_ref, g2_ref, be2_ref,
              w3_ref, b3_ref, out_ref):
    xn = xn_ref[...]
    m = jnp.mean(xn, axis=0, keepdims=True)
    v = jnp.mean((xn - m) * (xn - m), axis=0, keepdims=True)
    xn = gn_ref[...] * (xn - m) * lax.rsqrt(v + EPS) + bn_ref[...]

    h = jnp.dot(emb_ref[...], w1a_ref[...], preferred_element_type=jnp.float32)
    h = h + jnp.dot(xn, w1b_ref[...], preferred_element_type=jnp.float32)
    h = jnp.maximum(h + b1_ref[...], 0.0)
    m1 = jnp.mean(h, axis=0, keepdims=True)
    v1 = jnp.mean((h - m1) * (h - m1), axis=0, keepdims=True)
    h = g1_ref[...] * (h - m1) * lax.rsqrt(v1 + EPS) + be1_ref[...]

    h2 = jnp.dot(h, w2_ref[...], preferred_element_type=jnp.float32)
    h2 = jnp.maximum(h2 + b2_ref[...], 0.0)
    m2 = jnp.mean(h2, axis=0, keepdims=True)
    v2 = jnp.mean((h2 - m2) * (h2 - m2), axis=0, keepdims=True)
    h2 = g2_ref[...] * (h2 - m2) * lax.rsqrt(v2 + EPS) + be2_ref[...]

    out_ref[...] = (
        jnp.dot(h2, w3_ref[...], preferred_element_type=jnp.float32)
        + b3_ref[...]
    )


def kernel(x_categorical, x_numerical, emb_tables, bn_num_g, bn_num_b,
           W1, b1, g1, be1, W2, b2, g2, be2, W3, b3):
    xc = x_categorical.astype(jnp.int32)
    offs = (jnp.arange(F, dtype=jnp.int32) * (V // 4))[None, :]
    idx_blk = ((xc >> 2) + offs).reshape(_BT)
    table = emb_tables.reshape(_TBLK, 4 * D)
    emb4 = _make_sc_gather()(table, idx_blk)

    sel = (xc & 3).astype(jnp.int32)
    emb = pl.pallas_call(
        _select_body,
        grid=(B // _RSEL,),
        in_specs=[
            pl.BlockSpec((_RSEL, F * 128), lambda i: (i, 0)),
            pl.BlockSpec((_RSEL, F), lambda i: (i, 0)),
        ],
        out_specs=pl.BlockSpec((_RSEL, F * D), lambda i: (i, 0)),
        out_shape=jax.ShapeDtypeStruct((B, F * D), jnp.float32),
    )(emb4.reshape(B, F * 128), sel)

    xn = jnp.pad(x_numerical, ((0, 0), (0, NUMP - NUM)))
    gn = jnp.pad(bn_num_g, (0, NUMP - NUM)).reshape(1, NUMP)
    bn = jnp.pad(bn_num_b, (0, NUMP - NUM)).reshape(1, NUMP)
    w1a = W1[:, :F * D].T
    w1b = jnp.pad(W1[:, F * D:], ((0, 0), (0, NUMP - NUM))).T

    return pl.pallas_call(
        _mlp_body,
        out_shape=jax.ShapeDtypeStruct((B, OUT), jnp.float32),
    )(emb, xn, gn, bn, w1a, w1b, b1.reshape(1, H1),
      g1.reshape(1, H1), be1.reshape(1, H1), W2.T, b2.reshape(1, H2),
      g2.reshape(1, H2), be2.reshape(1, H2), W3.T, b3.reshape(1, OUT))


# speedup vs baseline: 1.0003x; 1.0003x over previous
import functools

import jax
import jax.numpy as jnp
from jax import lax
from jax.experimental import pallas as pl
from jax.experimental.pallas import tpu as pltpu
from jax.experimental.pallas import tpu_sc as plsc

B = 4096
F = 26
V = 100000
D = 32
NUM = 13
H1 = 512
H2 = 256
OUT = 100
EPS = 1e-5
NUMP = 128

_NC, _NS = 2, 16
_NW = _NC * _NS
_BT = B * F
_BPW = _BT // _NW
_CH = 256
_NCHUNK = _BPW // _CH
_TBLK = F * V // 4


@functools.cache
def _make_sc_gather():
    mesh = plsc.VectorSubcoreMesh(core_axis_name="c", subcore_axis_name="s")

    @functools.partial(
        pl.kernel,
        mesh=mesh,
        out_type=jax.ShapeDtypeStruct((_BT, 4 * D), jnp.float32),
        compiler_params=pltpu.CompilerParams(use_tc_tiling_on_sc=True),
        scratch_types=[
            pltpu.VMEM((_BPW,), jnp.int32),
            pltpu.VMEM((_CH, 4 * D), jnp.float32),
            pltpu.VMEM((_CH, 4 * D), jnp.float32),
            pltpu.SemaphoreType.DMA,
            pltpu.SemaphoreType.DMA,
            pltpu.SemaphoreType.DMA,
            pltpu.SemaphoreType.DMA,
        ],
    )
    def _sc_gather(table_hbm, idx_hbm, out_hbm, idx_v, r0, r1,
                   sg0, sg1, sw0, sw1):
        wid = lax.axis_index("s") * _NC + lax.axis_index("c")
        base = wid * _BPW
        pltpu.sync_copy(idx_hbm.at[pl.ds(base, _BPW)], idx_v)
        bufs, gsem, wsem = (r0, r1), (sg0, sg1), (sw0, sw1)
        wprev = [None, None]
        for c in range(_NCHUNK):
            b = c & 1
            if wprev[b] is not None:
                wprev[b].wait()
            g = pltpu.async_copy(
                table_hbm.at[idx_v.at[pl.ds(c * _CH, _CH)]], bufs[b], gsem[b])
            g.wait()
            wprev[b] = pltpu.async_copy(
                bufs[b], out_hbm.at[pl.ds(base + c * _CH, _CH)], wsem[b])
        wprev[0].wait()
        wprev[1].wait()

    return _sc_gather


_RSEL = 256


def _select_body(e_ref, s_ref, o_ref):
    e = e_ref[...]
    s = s_ref[...]
    outs = []
    for f in range(F):
        blk = e[:, f * 128:(f + 1) * 128]
        sf = s[:, f:f + 1]
        acc = blk[:, 0:32] * (sf == 0)
        for j in range(1, 4):
            acc = acc + blk[:, j * 32:(j + 1) * 32] * (sf == j)
        outs.append(acc)
    o_ref[...] = jnp.concatenate(outs, axis=1)


def _mlp_body(emb_ref, xn_ref, gn_ref, bn_ref, w1a_ref, w1b_ref, b1_ref,
              g1_ref, be1_ref, w2_ref, b2_ref, g2_ref, be2_ref,
              w3_ref, b3_ref, out_ref):
    xn = xn_ref[...]
    m = jnp.mean(xn, axis=0, keepdims=True)
    v = jnp.mean((xn - m) * (xn - m), axis=0, keepdims=True)
    xn = gn_ref[...] * (xn - m) * lax.rsqrt(v + EPS) + bn_ref[...]

    h = jnp.dot(emb_ref[...], w1a_ref[...], preferred_element_type=jnp.float32)
    h = h + jnp.dot(xn, w1b_ref[...], preferred_element_type=jnp.float32)
    h = jnp.maximum(h + b1_ref[...], 0.0)
    m1 = jnp.mean(h, axis=0, keepdims=True)
    v1 = jnp.mean((h - m1) * (h - m1), axis=0, keepdims=True)
    h = g1_ref[...] * (h - m1) * lax.rsqrt(v1 + EPS) + be1_ref[...]

    h2 = jnp.dot(h, w2_ref[...], preferred_element_type=jnp.float32)
    h2 = jnp.maximum(h2 + b2_ref[...], 0.0)
    m2 = jnp.mean(h2, axis=0, keepdims=True)
    v2 = jnp.mean((h2 - m2) * (h2 - m2), axis=0, keepdims=True)
    h2 = g2_ref[...] * (h2 - m2) * lax.rsqrt(v2 + EPS) + be2_ref[...]

    out_ref[...] = (
        jnp.dot(h2, w3_ref[...], preferred_element_type=jnp.float32)
        + b3_ref[...]
    )


def kernel(x_categorical, x_numerical, emb_tables, bn_num_g, bn_num_b,
           W1, b1, g1, be1, W2, b2, g2, be2, W3, b3):
    xc = x_categorical.astype(jnp.int32)
    offs = (jnp.arange(F, dtype=jnp.int32) * (V // 4))[None, :]
    idx_blk = ((xc >> 2) + offs).reshape(_BT)
    table = emb_tables.reshape(_TBLK, 4 * D)
    emb4 = _make_sc_gather()(table, idx_blk)

    sel = (xc & 3).astype(jnp.int32)
    emb = pl.pallas_call(
        _select_body,
        grid=(B // _RSEL,),
        in_specs=[
            pl.BlockSpec((_RSEL, F * 128), lambda i: (i, 0)),
            pl.BlockSpec((_RSEL, F), lambda i: (i, 0)),
        ],
        out_specs=pl.BlockSpec((_RSEL, F * D), lambda i: (i, 0)),
        out_shape=jax.ShapeDtypeStruct((B, F * D), jnp.float32),
    )(emb4.reshape(B, F * 128), sel)

    xn = jnp.pad(x_numerical, ((0, 0), (0, NUMP - NUM)))
    gn = jnp.pad(bn_num_g, (0, NUMP - NUM)).reshape(1, NUMP)
    bn = jnp.pad(bn_num_b, (0, NUMP - NUM)).reshape(1, NUMP)
    w1a = W1[:, :F * D].T
    w1b = jnp.pad(W1[:, F * D:], ((0, 0), (0, NUMP - NUM))).T

    return pl.pallas_call(
        _mlp_body,
        out_shape=jax.ShapeDtypeStruct((B, OUT), jnp.float32),
    )(emb, xn, gn, bn, w1a, w1b, b1.reshape(1, H1),
      g1.reshape(1, H1), be1.reshape(1, H1), W2.T, b2.reshape(1, H2),
      g2.reshape(1, H2), be2.reshape(1, H2), W3.T, b3.reshape(1, OUT))


# speedup vs baseline: 1.1013x; 1.1010x over previous
import functools

import jax
import jax.numpy as jnp
from jax import lax
from jax.experimental import pallas as pl
from jax.experimental.pallas import tpu as pltpu
from jax.experimental.pallas import tpu_sc as plsc

B = 4096
F = 26
V = 100000
D = 32
NUM = 13
H1 = 512
H2 = 256
OUT = 100
EPS = 1e-5
NUMP = 128

_NC, _NS = 2, 16
_NW = _NC * _NS
_BW = B // _NW


@functools.cache
def _make_sc_gather():
    mesh = plsc.VectorSubcoreMesh(core_axis_name="c", subcore_axis_name="s")

    @functools.partial(
        pl.kernel,
        mesh=mesh,
        out_type=jax.ShapeDtypeStruct((B, F, D), jnp.float32),
        compiler_params=pltpu.CompilerParams(use_tc_tiling_on_sc=False),
        scratch_types=[
            pltpu.VMEM((_BW,), jnp.int32),
            pltpu.VMEM((_BW,), jnp.int32),
            pltpu.VMEM((_BW, D), jnp.float32),
            pltpu.VMEM((_BW, D), jnp.float32),
            pltpu.SemaphoreType.DMA,
            pltpu.SemaphoreType.DMA,
            pltpu.SemaphoreType.DMA,
            pltpu.SemaphoreType.DMA,
        ],
    )
    def _sc_gather(table_hbm, xct_hbm, out_hbm, i0, i1, r0, r1,
                   sg0, sg1, sw0, sw1):
        wid = lax.axis_index("s") * _NC + lax.axis_index("c")
        b0 = wid * _BW
        ibufs, rbufs = (i0, i1), (r0, r1)
        gsem, wsem = (sg0, sg1), (sw0, sw1)
        wprev = [None, None]
        for f in range(F):
            p = f & 1
            if wprev[p] is not None:
                wprev[p].wait()
            pltpu.sync_copy(xct_hbm.at[f, pl.ds(b0, _BW)], ibufs[p])
            g = pltpu.async_copy(
                table_hbm.at[f].at[ibufs[p]], rbufs[p], gsem[p])
            g.wait()
            wprev[p] = pltpu.async_copy(
                rbufs[p], out_hbm.at[pl.ds(b0, _BW), f], wsem[p])
        wprev[0].wait()
        wprev[1].wait()

    return _sc_gather


def _mlp_body(emb_ref, xn_ref, gn_ref, bn_ref, w1a_ref, w1b_ref, b1_ref,
              g1_ref, be1_ref, w2_ref, b2_ref, g2_ref, be2_ref,
              w3_ref, b3_ref, out_ref):
    xn = xn_ref[...]
    m = jnp.mean(xn, axis=0, keepdims=True)
    v = jnp.mean((xn - m) * (xn - m), axis=0, keepdims=True)
    xn = gn_ref[...] * (xn - m) * lax.rsqrt(v + EPS) + bn_ref[...]

    h = jnp.dot(emb_ref[...], w1a_ref[...], preferred_element_type=jnp.float32)
    h = h + jnp.dot(xn, w1b_ref[...], preferred_element_type=jnp.float32)
    h = jnp.maximum(h + b1_ref[...], 0.0)
    m1 = jnp.mean(h, axis=0, keepdims=True)
    v1 = jnp.mean((h - m1) * (h - m1), axis=0, keepdims=True)
    h = g1_ref[...] * (h - m1) * lax.rsqrt(v1 + EPS) + be1_ref[...]

    h2 = jnp.dot(h, w2_ref[...], preferred_element_type=jnp.float32)
    h2 = jnp.maximum(h2 + b2_ref[...], 0.0)
    m2 = jnp.mean(h2, axis=0, keepdims=True)
    v2 = jnp.mean((h2 - m2) * (h2 - m2), axis=0, keepdims=True)
    h2 = g2_ref[...] * (h2 - m2) * lax.rsqrt(v2 + EPS) + be2_ref[...]

    out_ref[...] = (
        jnp.dot(h2, w3_ref[...], preferred_element_type=jnp.float32)
        + b3_ref[...]
    )


def kernel(x_categorical, x_numerical, emb_tables, bn_num_g, bn_num_b,
           W1, b1, g1, be1, W2, b2, g2, be2, W3, b3):
    xct = x_categorical.astype(jnp.int32).T
    emb = _make_sc_gather()(emb_tables, xct).reshape(B, F * D)

    xn = jnp.pad(x_numerical, ((0, 0), (0, NUMP - NUM)))
    gn = jnp.pad(bn_num_g, (0, NUMP - NUM)).reshape(1, NUMP)
    bn = jnp.pad(bn_num_b, (0, NUMP - NUM)).reshape(1, NUMP)
    w1a = W1[:, :F * D].T
    w1b = jnp.pad(W1[:, F * D:], ((0, 0), (0, NUMP - NUM))).T

    return pl.pallas_call(
        _mlp_body,
        out_shape=jax.ShapeDtypeStruct((B, OUT), jnp.float32),
    )(emb, xn, gn, bn, w1a, w1b, b1.reshape(1, H1),
      g1.reshape(1, H1), be1.reshape(1, H1), W2.T, b2.reshape(1, H2),
      g2.reshape(1, H2), be2.reshape(1, H2), W3.T, b3.reshape(1, OUT))


# speedup vs baseline: 1.1182x; 1.0153x over previous
import functools

import jax
import jax.numpy as jnp
from jax import lax
from jax.experimental import pallas as pl
from jax.experimental.pallas import tpu as pltpu
from jax.experimental.pallas import tpu_sc as plsc

B = 4096
F = 26
V = 100000
D = 32
NUM = 13
H1 = 512
H2 = 256
OUT = 100
EPS = 1e-5
NUMP = 128

_NC, _NS = 2, 16
_NW = _NC * _NS
_CH = 512
_NTASK = F * (B // _CH)
_TPW = -(-_NTASK // _NW)


@functools.cache
def _make_sc_gather():
    mesh = plsc.VectorSubcoreMesh(core_axis_name="c", subcore_axis_name="s")

    @functools.partial(
        pl.kernel,
        mesh=mesh,
        out_type=jax.ShapeDtypeStruct((B, F, D), jnp.float32),
        compiler_params=pltpu.CompilerParams(use_tc_tiling_on_sc=False),
        scratch_types=[
            pltpu.VMEM((_CH,), jnp.int32),
            pltpu.VMEM((_CH,), jnp.int32),
            pltpu.VMEM((_CH, D), jnp.float32),
            pltpu.VMEM((_CH, D), jnp.float32),
            pltpu.SemaphoreType.DMA,
            pltpu.SemaphoreType.DMA,
            pltpu.SemaphoreType.DMA,
            pltpu.SemaphoreType.DMA,
        ],
    )
    def _sc_gather(table_hbm, xct_hbm, out_hbm, i0, i1, r0, r1,
                   sg0, sg1, sw0, sw1):
        wid = lax.axis_index("s") * _NC + lax.axis_index("c")
        ibufs, rbufs = (i0, i1), (r0, r1)
        gsem, wsem = (sg0, sg1), (sw0, sw1)
        tpf = B // _CH
        nfull = _NTASK // _NW
        wprev = [None, None]
        for k in range(nfull):
            p = k & 1
            t = wid + k * _NW
            f = t // tpf
            b0 = (t % tpf) * _CH
            if wprev[p] is not None:
                wprev[p].wait()
            pltpu.sync_copy(xct_hbm.at[f, pl.ds(b0, _CH)], ibufs[p])
            pltpu.async_copy(
                table_hbm.at[f].at[ibufs[p]], rbufs[p], gsem[p]).wait()
            wprev[p] = pltpu.async_copy(
                rbufs[p], out_hbm.at[pl.ds(b0, _CH), f], wsem[p])
        wprev[0].wait()

        @pl.when(wid < _NTASK - nfull * _NW)
        def _():
            t = wid + nfull * _NW
            f = t // tpf
            b0 = (t % tpf) * _CH
            pltpu.sync_copy(xct_hbm.at[f, pl.ds(b0, _CH)], ibufs[0])
            pltpu.async_copy(
                table_hbm.at[f].at[ibufs[0]], rbufs[0], gsem[0]).wait()
            pltpu.async_copy(
                rbufs[0], out_hbm.at[pl.ds(b0, _CH), f], wsem[0]).wait()

        wprev[1].wait()

    return _sc_gather


def _mlp_body(emb_ref, xn_ref, gn_ref, bn_ref, w1a_ref, w1b_ref, b1_ref,
              g1_ref, be1_ref, w2_ref, b2_ref, g2_ref, be2_ref,
              w3_ref, b3_ref, out_ref):
    xn = xn_ref[...]
    m = jnp.mean(xn, axis=0, keepdims=True)
    v = jnp.mean((xn - m) * (xn - m), axis=0, keepdims=True)
    xn = gn_ref[...] * (xn - m) * lax.rsqrt(v + EPS) + bn_ref[...]

    h = jnp.dot(emb_ref[...], w1a_ref[...], preferred_element_type=jnp.float32)
    h = h + jnp.dot(xn, w1b_ref[...], preferred_element_type=jnp.float32)
    h = jnp.maximum(h + b1_ref[...], 0.0)
    m1 = jnp.mean(h, axis=0, keepdims=True)
    v1 = jnp.mean((h - m1) * (h - m1), axis=0, keepdims=True)
    h = g1_ref[...] * (h - m1) * lax.rsqrt(v1 + EPS) + be1_ref[...]

    h2 = jnp.dot(h, w2_ref[...], preferred_element_type=jnp.float32)
    h2 = jnp.maximum(h2 + b2_ref[...], 0.0)
    m2 = jnp.mean(h2, axis=0, keepdims=True)
    v2 = jnp.mean((h2 - m2) * (h2 - m2), axis=0, keepdims=True)
    h2 = g2_ref[...] * (h2 - m2) * lax.rsqrt(v2 + EPS) + be2_ref[...]

    out_ref[...] = (
        jnp.dot(h2, w3_ref[...], preferred_element_type=jnp.float32)
        + b3_ref[...]
    )


def kernel(x_categorical, x_numerical, emb_tables, bn_num_g, bn_num_b,
           W1, b1, g1, be1, W2, b2, g2, be2, W3, b3):
    xct = x_categorical.astype(jnp.int32).T
    emb = _make_sc_gather()(emb_tables, xct).reshape(B, F * D)

    xn = jnp.pad(x_numerical, ((0, 0), (0, NUMP - NUM)))
    gn = jnp.pad(bn_num_g, (0, NUMP - NUM)).reshape(1, NUMP)
    bn = jnp.pad(bn_num_b, (0, NUMP - NUM)).reshape(1, NUMP)
    w1a = W1[:, :F * D].T
    w1b = jnp.pad(W1[:, F * D:], ((0, 0), (0, NUMP - NUM))).T

    return pl.pallas_call(
        _mlp_body,
        out_shape=jax.ShapeDtypeStruct((B, OUT), jnp.float32),
    )(emb, xn, gn, bn, w1a, w1b, b1.reshape(1, H1),
      g1.reshape(1, H1), be1.reshape(1, H1), W2.T, b2.reshape(1, H2),
      g2.reshape(1, H2), be2.reshape(1, H2), W3.T, b3.reshape(1, OUT))


# speedup vs baseline: 1.1810x; 1.0562x over previous
import functools

import jax
import jax.numpy as jnp
from jax import lax
from jax.experimental import pallas as pl
from jax.experimental.pallas import tpu as pltpu
from jax.experimental.pallas import tpu_sc as plsc

B = 4096
F = 26
V = 100000
D = 32
NUM = 13
H1 = 512
H2 = 256
OUT = 100
EPS = 1e-5
NUMP = 128

_NC, _NS = 2, 16
_NW = _NC * _NS
_BT = B * F
_BPW = _BT // _NW

@functools.cache
def _make_sc_gather():
    mesh = plsc.VectorSubcoreMesh(
        core_axis_name="c", subcore_axis_name="s")

    @functools.partial(
        pl.kernel,
        mesh=mesh,
        out_type=jax.ShapeDtypeStruct((_BT, D), jnp.float32),
        compiler_params=pltpu.CompilerParams(use_tc_tiling_on_sc=False),
        scratch_types=[
            pltpu.VMEM((_BPW,), jnp.int32),
            pltpu.VMEM((_BPW, D), jnp.float32),
            pltpu.SemaphoreType.DMA,
        ],
    )
    def _sc_gather(table_hbm, idx_hbm, out_hbm, idx_v, rows_v, sem):
        wid = lax.axis_index("s") * _NC + lax.axis_index("c")
        base = wid * _BPW
        pltpu.sync_copy(idx_hbm.at[pl.ds(base, _BPW)], idx_v)
        pltpu.async_copy(table_hbm.at[idx_v], rows_v, sem).wait()
        pltpu.sync_copy(rows_v, out_hbm.at[pl.ds(base, _BPW)])

    return _sc_gather


def _mlp_body(emb_ref, xn_ref, gn_ref, bn_ref, w1a_ref, w1b_ref, b1_ref,
              g1_ref, be1_ref, w2_ref, b2_ref, g2_ref, be2_ref,
              w3_ref, b3_ref, out_ref):
    xn = xn_ref[...]
    m = jnp.mean(xn, axis=0, keepdims=True)
    v = jnp.mean((xn - m) * (xn - m), axis=0, keepdims=True)
    xn = gn_ref[...] * (xn - m) * lax.rsqrt(v + EPS) + bn_ref[...]

    h = jnp.dot(emb_ref[...], w1a_ref[...], preferred_element_type=jnp.float32)
    h = h + jnp.dot(xn, w1b_ref[...], preferred_element_type=jnp.float32)
    h = jnp.maximum(h + b1_ref[...], 0.0)
    m1 = jnp.mean(h, axis=0, keepdims=True)
    v1 = jnp.mean((h - m1) * (h - m1), axis=0, keepdims=True)
    h = g1_ref[...] * (h - m1) * lax.rsqrt(v1 + EPS) + be1_ref[...]

    h2 = jnp.dot(h, w2_ref[...], preferred_element_type=jnp.float32)
    h2 = jnp.maximum(h2 + b2_ref[...], 0.0)
    m2 = jnp.mean(h2, axis=0, keepdims=True)
    v2 = jnp.mean((h2 - m2) * (h2 - m2), axis=0, keepdims=True)
    h2 = g2_ref[...] * (h2 - m2) * lax.rsqrt(v2 + EPS) + be2_ref[...]

    out_ref[...] = (
        jnp.dot(h2, w3_ref[...], preferred_element_type=jnp.float32)
        + b3_ref[...]
    )


def kernel(x_categorical, x_numerical, emb_tables, bn_num_g, bn_num_b,
           W1, b1, g1, be1, W2, b2, g2, be2, W3, b3):
    offs = (jnp.arange(F, dtype=jnp.int32) * V)[None, :]
    idx = (x_categorical.astype(jnp.int32) + offs).reshape(_BT)
    table = emb_tables.reshape(F * V, D)
    emb = _make_sc_gather()(table, idx).reshape(B, F * D)

    xn = jnp.pad(x_numerical, ((0, 0), (0, NUMP - NUM)))
    gn = jnp.pad(bn_num_g, (0, NUMP - NUM)).reshape(1, NUMP)
    bn = jnp.pad(bn_num_b, (0, NUMP - NUM)).reshape(1, NUMP)
    w1a = W1[:, :F * D].T
    w1b = jnp.pad(W1[:, F * D:], ((0, 0), (0, NUMP - NUM))).T

    return pl.pallas_call(
        _mlp_body,
        out_shape=jax.ShapeDtypeStruct((B, OUT), jnp.float32),
    )(emb, xn, gn, bn, w1a, w1b, b1.reshape(1, H1),
      g1.reshape(1, H1), be1.reshape(1, H1), W2.T, b2.reshape(1, H2),
      g2.reshape(1, H2), be2.reshape(1, H2), W3.T, b3.reshape(1, OUT))
